# baseline (speedup 1.0000x reference)
import functools

import jax
import jax.numpy as jnp
from jax.experimental import pallas as pl
from jax.experimental.pallas import tpu as pltpu


def _hard_sigmoid(x):
    return jnp.clip(0.2 * x + 0.5, 0.0, 1.0)


def _mann_body(Ts, T, b, u, halves,
               inputs_ref, h_tm1_ref, c_tm1_ref, r_tm1_ref, w_ref, rk_ref,
               b_ref, wg_ref, mem_ref, wu_ref,
               h_out, c_out, r_out, memnew_out, wunew_out,
               mem_copy, e_s, wu_copy, keynT_s, h_bf_s, l_s, minv_s, mini_s):
    phase = pl.program_id(0)
    t = pl.program_id(1)
    sub = Ts // halves

    @pl.when((phase == 0) & (t == 0))
    def _lstm():
        x = jnp.dot(inputs_ref[...], w_ref[...],
                    preferred_element_type=jnp.float32) + b_ref[...]
        rk = rk_ref[...]
        hr = jnp.dot(h_tm1_ref[...], rk[:, :4 * u],
                     preferred_element_type=jnp.float32)
        rr = jnp.dot(r_tm1_ref[...], rk[:, 4 * u:],
                     preferred_element_type=jnp.float32)
        i = _hard_sigmoid(x[:, :u] + hr[:, :u] + rr)
        f = _hard_sigmoid(x[:, u:2 * u] + hr[:, u:2 * u])
        c = f * c_tm1_ref[...] + i * jnp.tanh(x[:, 2 * u:3 * u] + hr[:, 2 * u:3 * u])
        o = _hard_sigmoid(x[:, 3 * u:] + hr[:, 3 * u:])
        h = o * jnp.tanh(c)
        h_out[...] = h
        c_out[...] = c
        h_bf_s[...] = h.astype(jnp.bfloat16)
        nrm = jnp.sqrt(jnp.sum(h * h, axis=1, keepdims=True))
        keynT_s[...] = jnp.transpose(h / (nrm + 1e-8)).astype(jnp.bfloat16)
        l_s[...] = jnp.zeros((b, 128), jnp.float32)
        minv_s[...] = jnp.full((b, 128), jnp.inf, jnp.float32)
        mini_s[...] = jnp.zeros((b, 128), jnp.int32)

    @pl.when(phase == 0)
    def _p0():
        keynT = keynT_s[...]
        ones_u = jnp.ones((u, b), jnp.bfloat16)
        lsums = []
        for j in range(halves):
            mem_bf = mem_ref[j * sub:(j + 1) * sub, :].astype(jnp.bfloat16)
            mem_copy[pl.ds(t * Ts + j * sub, sub), :] = mem_bf
            simt = jnp.dot(mem_bf, keynT,
                           preferred_element_type=jnp.float32)
            ssqt = jnp.dot(mem_bf * mem_bf, ones_u,
                           preferred_element_type=jnp.float32)
            sim_row = jnp.transpose(simt)
            ssq_row = jnp.transpose(ssqt)[0:1, :]
            rinv = 1.0 / (jnp.sqrt(ssq_row) + 1e-8)
            e = jnp.exp(sim_row * rinv)
            e_s[:, pl.ds(t * Ts + j * sub, sub)] = e
            lsums.append(jnp.sum(e, axis=1, keepdims=True))
        l_s[...] = l_s[...] + jnp.broadcast_to(sum(lsums), (b, 128))
        wu_t = wu_ref[...]
        wu_copy[:, pl.ds(t * Ts, Ts)] = wu_t
        tmin = jnp.min(wu_t, axis=1, keepdims=True)
        lanes = jax.lax.broadcasted_iota(jnp.int32, (b, Ts), 1)
        tidx = jnp.min(jnp.where(wu_t == tmin, lanes, jnp.int32(2 ** 30)),
                       axis=1, keepdims=True) + t * Ts
        better = tmin < minv_s[:, 0:1]
        mini_s[...] = jnp.broadcast_to(
            jnp.where(better, tidx, mini_s[:, 0:1]), (b, 128))
        minv_s[...] = jnp.broadcast_to(
            jnp.where(better, tmin, minv_s[:, 0:1]), (b, 128))

    @pl.when(phase == 1)
    def _p1():
        inv_l = 1.0 / l_s[:, 0:1]
        lu = mini_s[:, 0:1]
        sg = 1.0 / (1.0 + jnp.exp(-wg_ref[...]))
        h_bf = h_bf_s[...]
        ones_b = jnp.ones((b, 128), jnp.bfloat16)
        rcs = []
        for j in range(halves):
            mem_bf = mem_copy[pl.ds(t * Ts + j * sub, sub), :]
            e = e_s[:, pl.ds(t * Ts + j * sub, sub)]
            wr = e * inv_l
            rcs.append(jnp.dot(e.astype(jnp.bfloat16), mem_bf,
                               preferred_element_type=jnp.float32) * inv_l)
            lanes = (jax.lax.broadcasted_iota(jnp.int32, (b, sub), 1)
                     + (t * Ts + j * sub))
            wlu = (lanes == lu).astype(jnp.float32)
            ww = sg * wr + (1.0 - sg) * wlu
            q = jax.lax.dot_general(wlu.astype(jnp.bfloat16), ones_b,
                                    (((0,), (0,)), ((), ())),
                                    preferred_element_type=jnp.float32)
            upd = jax.lax.dot_general(ww.astype(jnp.bfloat16), h_bf,
                                      (((0,), (0,)), ((), ())),
                                      preferred_element_type=jnp.float32)
            memnew_out[j * sub:(j + 1) * sub, :] = jnp.where(
                q > 0.0, upd, mem_bf.astype(jnp.float32) + upd)
            wunew_out[:, j * sub:(j + 1) * sub] = (
                0.5 * wu_copy[:, pl.ds(t * Ts + j * sub, sub)] + wr + ww)
        rc = sum(rcs)

        @pl.when(t == 0)
        def _():
            r_out[...] = rc

        @pl.when(t != 0)
        def _():
            r_out[...] = r_out[...] + rc


def kernel(inputs, h_tm1, c_tm1, r_tm1, kernel, recurrent_kernel, bias,
           write_gate, memory, wu):
    n_slots, u = memory.shape
    b = inputs.shape[0]
    if n_slots % 4096 == 0:
        Ts, halves = 4096, 2
    else:
        Ts, halves = n_slots, 1
    T = n_slots // Ts
    bias2 = bias.reshape(1, 4 * u)
    wg2 = write_gate.reshape(1, 1)

    const = lambda p, t: (0, 0)
    outs = pl.pallas_call(
        functools.partial(_mann_body, Ts, T, b, u, halves),
        grid=(2, T),
        in_specs=[
            pl.BlockSpec(inputs.shape, const),
            pl.BlockSpec(h_tm1.shape, const),
            pl.BlockSpec(c_tm1.shape, const),
            pl.BlockSpec(r_tm1.shape, const),
            pl.BlockSpec(kernel.shape, const),
            pl.BlockSpec(recurrent_kernel.shape, const),
            pl.BlockSpec((1, 4 * u), const),
            pl.BlockSpec((1, 1), const),
            pl.BlockSpec((Ts, u), lambda p, t: (jnp.where(p == 0, t, T - 1), 0)),
            pl.BlockSpec((b, Ts), lambda p, t: (0, jnp.where(p == 0, t, T - 1))),
        ],
        out_specs=[
            pl.BlockSpec((b, u), const),
            pl.BlockSpec((b, u), const),
            pl.BlockSpec((b, u), const),
            pl.BlockSpec((Ts, u), lambda p, t: (jnp.where(p == 0, 0, t), 0)),
            pl.BlockSpec((b, Ts), lambda p, t: (0, jnp.where(p == 0, 0, t))),
        ],
        out_shape=[
            jax.ShapeDtypeStruct((b, u), jnp.float32),
            jax.ShapeDtypeStruct((b, u), jnp.float32),
            jax.ShapeDtypeStruct((b, u), jnp.float32),
            jax.ShapeDtypeStruct((n_slots, u), jnp.float32),
            jax.ShapeDtypeStruct((b, n_slots), jnp.float32),
        ],
        scratch_shapes=[
            pltpu.VMEM((n_slots, u), jnp.bfloat16),
            pltpu.VMEM((b, n_slots), jnp.float32),
            pltpu.VMEM((b, n_slots), jnp.float32),
            pltpu.VMEM((u, b), jnp.bfloat16),
            pltpu.VMEM((b, u), jnp.bfloat16),
            pltpu.VMEM((b, 128), jnp.float32),
            pltpu.VMEM((b, 128), jnp.float32),
            pltpu.VMEM((b, 128), jnp.int32),
        ],
    )(inputs, h_tm1, c_tm1, r_tm1, kernel, recurrent_kernel, bias2, wg2,
      memory, wu)
    h, c, r, mem_new, wu_new = outs
    return h, c, r, mem_new, wu_new


# speedup vs baseline: 1.0039x; 1.0039x over previous
import functools

import jax
import jax.numpy as jnp
from jax.experimental import pallas as pl
from jax.experimental.pallas import tpu as pltpu


def _hard_sigmoid(x):
    return jnp.clip(0.2 * x + 0.5, 0.0, 1.0)


def _mann_body(Ts, T, b, u, halves,
               inputs_ref, h_tm1_ref, c_tm1_ref, r_tm1_ref, w_ref, rk_ref,
               b_ref, wg_ref, mem_ref, wu_ref,
               h_out, c_out, r_out, memnew_out, wunew_out,
               mem_copy, e_s, wu_copy, keynT_s, h_bf_s, l_s, minv_s, mini_s):
    phase = pl.program_id(0)
    t = pl.program_id(1)
    sub = Ts // halves

    @pl.when((phase == 0) & (t == 0))
    def _lstm():
        x = jnp.dot(inputs_ref[...], w_ref[...],
                    preferred_element_type=jnp.float32) + b_ref[...]
        rk = rk_ref[...]
        hr = jnp.dot(h_tm1_ref[...], rk[:, :4 * u],
                     preferred_element_type=jnp.float32)
        rr = jnp.dot(r_tm1_ref[...], rk[:, 4 * u:],
                     preferred_element_type=jnp.float32)
        i = _hard_sigmoid(x[:, :u] + hr[:, :u] + rr)
        f = _hard_sigmoid(x[:, u:2 * u] + hr[:, u:2 * u])
        c = f * c_tm1_ref[...] + i * jnp.tanh(x[:, 2 * u:3 * u] + hr[:, 2 * u:3 * u])
        o = _hard_sigmoid(x[:, 3 * u:] + hr[:, 3 * u:])
        h = o * jnp.tanh(c)
        h_out[...] = h
        c_out[...] = c
        h_bf_s[...] = h.astype(jnp.bfloat16)
        nrm = jnp.sqrt(jnp.sum(h * h, axis=1, keepdims=True))
        keynT_s[...] = jnp.transpose(h / (nrm + 1e-8)).astype(jnp.bfloat16)
        l_s[...] = jnp.zeros((b, 128), jnp.float32)
        minv_s[...] = jnp.full((b, 128), jnp.inf, jnp.float32)
        mini_s[...] = jnp.zeros((b, 128), jnp.int32)

    @pl.when(phase == 0)
    def _p0():
        keynT = keynT_s[...]
        ones_u = jnp.ones((u, b), jnp.bfloat16)
        lsums = []
        for j in range(halves):
            mem_bf = mem_ref[j * sub:(j + 1) * sub, :].astype(jnp.bfloat16)
            mem_copy[pl.ds(t * Ts + j * sub, sub), :] = mem_bf
            simt = jnp.dot(mem_bf, keynT,
                           preferred_element_type=jnp.float32)
            ssqt = jnp.dot(mem_bf * mem_bf, ones_u,
                           preferred_element_type=jnp.float32)
            sim_row = jnp.transpose(simt)
            ssq_row = jnp.transpose(ssqt)[0:1, :]
            rinv = 1.0 / (jnp.sqrt(ssq_row) + 1e-8)
            e = jnp.exp(sim_row * rinv)
            e_s[:, pl.ds(t * Ts + j * sub, sub)] = e
            lsums.append(jnp.sum(e, axis=1, keepdims=True))
        l_s[...] = l_s[...] + jnp.broadcast_to(sum(lsums), (b, 128))
        wu_t = wu_ref[...]
        wu_copy[:, pl.ds(t * Ts, Ts)] = wu_t
        tmin = jnp.min(wu_t, axis=1, keepdims=True)
        lanes = jax.lax.broadcasted_iota(jnp.int32, (b, Ts), 1)
        tidx = jnp.min(jnp.where(wu_t == tmin, lanes, jnp.int32(2 ** 30)),
                       axis=1, keepdims=True) + t * Ts
        better = tmin < minv_s[:, 0:1]
        mini_s[...] = jnp.broadcast_to(
            jnp.where(better, tidx, mini_s[:, 0:1]), (b, 128))
        minv_s[...] = jnp.broadcast_to(
            jnp.where(better, tmin, minv_s[:, 0:1]), (b, 128))

    @pl.when(phase == 1)
    def _p1():
        inv_l = 1.0 / l_s[:, 0:1]
        lu = mini_s[:, 0:1]
        sg = 1.0 / (1.0 + jnp.exp(-wg_ref[...]))
        h_bf = h_bf_s[...]
        ones_b = jnp.ones((b, 128), jnp.bfloat16)
        rcs = []
        for j in range(halves):
            mem_bf = mem_copy[pl.ds(t * Ts + j * sub, sub), :]
            e = e_s[:, pl.ds(t * Ts + j * sub, sub)]
            wr = e * inv_l
            rcs.append(jnp.dot(e.astype(jnp.bfloat16), mem_bf,
                               preferred_element_type=jnp.float32) * inv_l)
            lanes = (jax.lax.broadcasted_iota(jnp.int32, (b, sub), 1)
                     + (t * Ts + j * sub))
            wlu = (lanes == lu).astype(jnp.float32)
            ww = sg * wr + (1.0 - sg) * wlu
            q = jax.lax.dot_general(wlu.astype(jnp.bfloat16), ones_b,
                                    (((0,), (0,)), ((), ())),
                                    preferred_element_type=jnp.float32)
            upd = jax.lax.dot_general(ww.astype(jnp.bfloat16), h_bf,
                                      (((0,), (0,)), ((), ())),
                                      preferred_element_type=jnp.float32)
            memnew_out[j * sub:(j + 1) * sub, :] = jnp.where(
                q > 0.0, upd, mem_bf.astype(jnp.float32) + upd)
            wunew_out[:, j * sub:(j + 1) * sub] = (
                0.5 * wu_copy[:, pl.ds(t * Ts + j * sub, sub)] + wr + ww)
        rc = sum(rcs)

        @pl.when(t == 0)
        def _():
            r_out[...] = rc

        @pl.when(t != 0)
        def _():
            r_out[...] = r_out[...] + rc


def kernel(inputs, h_tm1, c_tm1, r_tm1, kernel, recurrent_kernel, bias,
           write_gate, memory, wu):
    n_slots, u = memory.shape
    b = inputs.shape[0]
    if n_slots % 4096 == 0:
        Ts, halves = 4096, 4
    else:
        Ts, halves = n_slots, 1
    T = n_slots // Ts
    bias2 = bias.reshape(1, 4 * u)
    wg2 = write_gate.reshape(1, 1)

    const = lambda p, t: (0, 0)
    outs = pl.pallas_call(
        functools.partial(_mann_body, Ts, T, b, u, halves),
        grid=(2, T),
        in_specs=[
            pl.BlockSpec(inputs.shape, const),
            pl.BlockSpec(h_tm1.shape, const),
            pl.BlockSpec(c_tm1.shape, const),
            pl.BlockSpec(r_tm1.shape, const),
            pl.BlockSpec(kernel.shape, const),
            pl.BlockSpec(recurrent_kernel.shape, const),
            pl.BlockSpec((1, 4 * u), const),
            pl.BlockSpec((1, 1), const),
            pl.BlockSpec((Ts, u), lambda p, t: (jnp.where(p == 0, t, T - 1), 0)),
            pl.BlockSpec((b, Ts), lambda p, t: (0, jnp.where(p == 0, t, T - 1))),
        ],
        out_specs=[
            pl.BlockSpec((b, u), const),
            pl.BlockSpec((b, u), const),
            pl.BlockSpec((b, u), const),
            pl.BlockSpec((Ts, u), lambda p, t: (jnp.where(p == 0, 0, t), 0)),
            pl.BlockSpec((b, Ts), lambda p, t: (0, jnp.where(p == 0, 0, t))),
        ],
        out_shape=[
            jax.ShapeDtypeStruct((b, u), jnp.float32),
            jax.ShapeDtypeStruct((b, u), jnp.float32),
            jax.ShapeDtypeStruct((b, u), jnp.float32),
            jax.ShapeDtypeStruct((n_slots, u), jnp.float32),
            jax.ShapeDtypeStruct((b, n_slots), jnp.float32),
        ],
        scratch_shapes=[
            pltpu.VMEM((n_slots, u), jnp.bfloat16),
            pltpu.VMEM((b, n_slots), jnp.float32),
            pltpu.VMEM((b, n_slots), jnp.float32),
            pltpu.VMEM((u, b), jnp.bfloat16),
            pltpu.VMEM((b, u), jnp.bfloat16),
            pltpu.VMEM((b, 128), jnp.float32),
            pltpu.VMEM((b, 128), jnp.float32),
            pltpu.VMEM((b, 128), jnp.int32),
        ],
    )(inputs, h_tm1, c_tm1, r_tm1, kernel, recurrent_kernel, bias2, wg2,
      memory, wu)
    h, c, r, mem_new, wu_new = outs
    return h, c, r, mem_new, wu_new


# speedup vs baseline: 1.1609x; 1.1564x over previous
import functools

import jax
import jax.numpy as jnp
from jax.experimental import pallas as pl
from jax.experimental.pallas import tpu as pltpu


def _hard_sigmoid(x):
    return jnp.clip(0.2 * x + 0.5, 0.0, 1.0)


def _mann_body(Ts, T, b, u, halves,
               inputs_ref, h_tm1_ref, c_tm1_ref, r_tm1_ref, w_ref, rk_ref,
               b_ref, wg_ref, mem_ref, wu_ref,
               h_out, c_out, r_out, memnew_out, wunew_out,
               mem_copy, e_s, wu_copy, keynT_s, h_bf_s, l_s, minv_s, mini_s):
    phase = pl.program_id(0)
    t = pl.program_id(1)
    sub = Ts // halves

    @pl.when((phase == 0) & (t == 0))
    def _lstm():
        x = jnp.dot(inputs_ref[...], w_ref[...],
                    preferred_element_type=jnp.float32) + b_ref[...]
        rk = rk_ref[...]
        hr = jnp.dot(h_tm1_ref[...], rk[:, :4 * u],
                     preferred_element_type=jnp.float32)
        rr = jnp.dot(r_tm1_ref[...], rk[:, 4 * u:],
                     preferred_element_type=jnp.float32)
        i = _hard_sigmoid(x[:, :u] + hr[:, :u] + rr)
        f = _hard_sigmoid(x[:, u:2 * u] + hr[:, u:2 * u])
        c = f * c_tm1_ref[...] + i * jnp.tanh(x[:, 2 * u:3 * u] + hr[:, 2 * u:3 * u])
        o = _hard_sigmoid(x[:, 3 * u:] + hr[:, 3 * u:])
        h = o * jnp.tanh(c)
        h_out[...] = h
        c_out[...] = c
        h_bf_s[...] = h.astype(jnp.bfloat16)
        nrm = jnp.sqrt(jnp.sum(h * h, axis=1, keepdims=True))
        keynT_s[...] = jnp.transpose(h / (nrm + 1e-8)).astype(jnp.bfloat16)
        l_s[...] = jnp.zeros((b, 128), jnp.float32)
        minv_s[...] = jnp.full((b, 128), jnp.inf, jnp.float32)
        mini_s[...] = jnp.zeros((b, 128), jnp.int32)

    @pl.when(phase == 0)
    def _p0():
        keynT = keynT_s[...]
        ones_u = jnp.ones((u, b), jnp.bfloat16)
        lsums = []
        for j in range(halves):
            mem_bf = mem_ref[j * sub:(j + 1) * sub, :].astype(jnp.bfloat16)
            mem_copy[pl.ds(t * Ts + j * sub, sub), :] = mem_bf
            simt = jnp.dot(mem_bf, keynT,
                           preferred_element_type=jnp.float32)
            ssqt = jnp.dot(mem_bf * mem_bf, ones_u,
                           preferred_element_type=jnp.float32)
            sim_row = jnp.transpose(simt)
            ssq_row = jnp.transpose(ssqt)[0:1, :]
            rinv = 1.0 / (jnp.sqrt(ssq_row) + 1e-8)
            e = jnp.exp(sim_row * rinv)
            e_s[:, pl.ds(t * Ts + j * sub, sub)] = e
            lsums.append(jnp.sum(e, axis=1, keepdims=True))
        l_s[...] = l_s[...] + jnp.broadcast_to(sum(lsums), (b, 128))
        wu_t = wu_ref[...]
        wu_copy[:, pl.ds(t * Ts, Ts)] = wu_t
        tmin = jnp.min(wu_t, axis=1, keepdims=True)
        lanes = jax.lax.broadcasted_iota(jnp.int32, (b, Ts), 1)
        tidx = jnp.min(jnp.where(wu_t == tmin, lanes, jnp.int32(2 ** 30)),
                       axis=1, keepdims=True) + t * Ts
        better = tmin < minv_s[:, 0:1]
        mini_s[...] = jnp.broadcast_to(
            jnp.where(better, tidx, mini_s[:, 0:1]), (b, 128))
        minv_s[...] = jnp.broadcast_to(
            jnp.where(better, tmin, minv_s[:, 0:1]), (b, 128))

    @pl.when(phase == 1)
    def _p1():
        inv_l = 1.0 / l_s[:, 0:1]
        lu = mini_s[:, 0:1]
        sg = 1.0 / (1.0 + jnp.exp(-wg_ref[...]))
        h_bf = h_bf_s[...]
        ones_b = jnp.ones((b, 128), jnp.bfloat16)
        rcs = []
        for j in range(halves):
            mem_bf = mem_copy[pl.ds(t * Ts + j * sub, sub), :]
            e = e_s[:, pl.ds(t * Ts + j * sub, sub)]
            wr = e * inv_l
            rcs.append(jnp.dot(e.astype(jnp.bfloat16), mem_bf,
                               preferred_element_type=jnp.float32) * inv_l)
            lanes = (jax.lax.broadcasted_iota(jnp.int32, (b, sub), 1)
                     + (t * Ts + j * sub))
            wlu = (lanes == lu).astype(jnp.float32)
            ww = sg * wr + (1.0 - sg) * wlu
            q = jax.lax.dot_general(wlu.astype(jnp.bfloat16), ones_b,
                                    (((0,), (0,)), ((), ())),
                                    preferred_element_type=jnp.float32)
            upd = jax.lax.dot_general(ww.astype(jnp.bfloat16), h_bf,
                                      (((0,), (0,)), ((), ())),
                                      preferred_element_type=jnp.float32)
            memnew_out[j * sub:(j + 1) * sub, :] = jnp.where(
                q > 0.0, upd, mem_bf.astype(jnp.float32) + upd)
            wunew_out[:, j * sub:(j + 1) * sub] = (
                0.5 * wu_copy[:, pl.ds(t * Ts + j * sub, sub)] + wr + ww)
        rc = sum(rcs)

        @pl.when(t == 0)
        def _():
            r_out[...] = rc

        @pl.when(t != 0)
        def _():
            r_out[...] = r_out[...] + rc


def kernel(inputs, h_tm1, c_tm1, r_tm1, kernel, recurrent_kernel, bias,
           write_gate, memory, wu):
    n_slots, u = memory.shape
    b = inputs.shape[0]
    if n_slots % 8192 == 0:
        Ts, halves = 8192, 2
    else:
        Ts, halves = n_slots, 1
    T = n_slots // Ts
    bias2 = bias.reshape(1, 4 * u)
    wg2 = write_gate.reshape(1, 1)

    const = lambda p, t: (0, 0)
    outs = pl.pallas_call(
        functools.partial(_mann_body, Ts, T, b, u, halves),
        grid=(2, T),
        in_specs=[
            pl.BlockSpec(inputs.shape, const),
            pl.BlockSpec(h_tm1.shape, const),
            pl.BlockSpec(c_tm1.shape, const),
            pl.BlockSpec(r_tm1.shape, const),
            pl.BlockSpec(kernel.shape, const),
            pl.BlockSpec(recurrent_kernel.shape, const),
            pl.BlockSpec((1, 4 * u), const),
            pl.BlockSpec((1, 1), const),
            pl.BlockSpec((Ts, u), lambda p, t: (jnp.where(p == 0, t, T - 1), 0)),
            pl.BlockSpec((b, Ts), lambda p, t: (0, jnp.where(p == 0, t, T - 1))),
        ],
        out_specs=[
            pl.BlockSpec((b, u), const),
            pl.BlockSpec((b, u), const),
            pl.BlockSpec((b, u), const),
            pl.BlockSpec((Ts, u), lambda p, t: (jnp.where(p == 0, 0, t), 0)),
            pl.BlockSpec((b, Ts), lambda p, t: (0, jnp.where(p == 0, 0, t))),
        ],
        out_shape=[
            jax.ShapeDtypeStruct((b, u), jnp.float32),
            jax.ShapeDtypeStruct((b, u), jnp.float32),
            jax.ShapeDtypeStruct((b, u), jnp.float32),
            jax.ShapeDtypeStruct((n_slots, u), jnp.float32),
            jax.ShapeDtypeStruct((b, n_slots), jnp.float32),
        ],
        scratch_shapes=[
            pltpu.VMEM((n_slots, u), jnp.bfloat16),
            pltpu.VMEM((b, n_slots), jnp.float32),
            pltpu.VMEM((b, n_slots), jnp.float32),
            pltpu.VMEM((u, b), jnp.bfloat16),
            pltpu.VMEM((b, u), jnp.bfloat16),
            pltpu.VMEM((b, 128), jnp.float32),
            pltpu.VMEM((b, 128), jnp.float32),
            pltpu.VMEM((b, 128), jnp.int32),
        ],
    )(inputs, h_tm1, c_tm1, r_tm1, kernel, recurrent_kernel, bias2, wg2,
      memory, wu)
    h, c, r, mem_new, wu_new = outs
    return h, c, r, mem_new, wu_new
